# baseline (speedup 1.0000x reference)
import functools

import jax
import jax.numpy as jnp
from jax import lax
from jax.experimental import pallas as pl
from jax.experimental.pallas import tpu as pltpu
from jax.experimental.pallas import tpu_sc as plsc

N = 10000
E = 320000
D = 128

_INFO = plsc.get_sparse_core_info()
NC = _INFO.num_cores
NS = _INFO.num_subcores
NW = NC * NS
CH = 64
SB = 27
NSB = 6
EPT = NSB * SB * CH
RPT = N // NS
NBUF = 4


def _scale_rows(buf, w_v, g):
    def row_body(r, c):
        q16 = (r // 16) * 16
        wrow = w_v[g, pl.ds(q16, 16)]
        idx = (jnp.zeros((16,), jnp.int32) + (r - q16))[:, None]
        wv = lax.gather(
            wrow, idx,
            lax.GatherDimensionNumbers(offset_dims=(),
                                       collapsed_slice_dims=(0,),
                                       start_index_map=(0,)),
            slice_sizes=(1,),
            mode=lax.GatherScatterMode.PROMISE_IN_BOUNDS)
        for j in range(D // 16):
            sl = (r, pl.ds(j * 16, 16))
            buf[sl] = buf[sl] * wv
        return c

    lax.fori_loop(0, CH, row_body, 0, unroll=False)


def _sc_aggregate(x, src3, dst3, w3, zeros):
    mesh = plsc.VectorSubcoreMesh(core_axis_name="c", subcore_axis_name="s")

    @functools.partial(
        pl.kernel,
        mesh=mesh,
        out_type=jax.ShapeDtypeStruct((NC, NS, RPT, D), jnp.float32),
        scratch_types=[
            pltpu.VMEM((SB, CH), jnp.int32),
            pltpu.VMEM((SB, CH), jnp.int32),
            pltpu.VMEM((SB, CH), jnp.float32),
            pltpu.VMEM((CH, D), jnp.float32),
            pltpu.VMEM((CH, D), jnp.float32),
            pltpu.VMEM((CH, D), jnp.float32),
            pltpu.VMEM((CH, D), jnp.float32),
            pltpu.VMEM_SHARED((N, D), jnp.float32),
            pltpu.SemaphoreType.DMA,
            pltpu.SemaphoreType.DMA,
            pltpu.SemaphoreType.DMA,
            pltpu.SemaphoreType.DMA,
            pltpu.SemaphoreType.DMA,
            pltpu.SemaphoreType.DMA,
            pltpu.SemaphoreType.DMA,
            pltpu.SemaphoreType.DMA,
        ],
    )
    def agg(x_hbm, src_hbm, dst_hbm, w_hbm, zeros_hbm, out_hbm,
            src_v, dst_v, w_v, buf0, buf1, buf2, buf3, acc,
            g0, g1, g2, g3, s0, s1, s2, s3):
        cid = lax.axis_index("c")
        sid = lax.axis_index("s")
        wid = sid * NC + cid
        bufs = (buf0, buf1, buf2, buf3)
        gsems = (g0, g1, g2, g3)
        ssems = (s0, s1, s2, s3)

        pltpu.sync_copy(zeros_hbm, acc.at[pl.ds(sid * RPT, RPT)])
        plsc.subcore_barrier()

        def super_body(sb, carry0):
            pltpu.sync_copy(src_hbm.at[wid, sb], src_v)
            pltpu.sync_copy(dst_hbm.at[wid, sb], dst_v)
            pltpu.sync_copy(w_hbm.at[wid, sb], w_v)

            gd = [None] * SB
            sd = [None] * SB
            for g in range(2):
                gd[g] = pltpu.async_copy(
                    x_hbm.at[src_v.at[g]], bufs[g % NBUF], gsems[g % NBUF])

            for g in range(SB):
                p = g % NBUF
                gd[g].wait()
                _scale_rows(bufs[p], w_v, g)
                if g + 2 < SB:
                    if g >= NBUF - 2:
                        sd[g - (NBUF - 2)].wait()
                    pn = (g + 2) % NBUF
                    gd[g + 2] = pltpu.async_copy(
                        x_hbm.at[src_v.at[g + 2]], bufs[pn], gsems[pn])
                sd[g] = pltpu.async_copy(
                    bufs[p], acc.at[dst_v.at[g]], ssems[p], add=True)
            for g in range(max(0, SB - NBUF), SB):
                sd[g].wait()
            return carry0

        lax.fori_loop(0, NSB, super_body, 0, unroll=False)

        plsc.subcore_barrier()
        pltpu.sync_copy(acc.at[pl.ds(sid * RPT, RPT)],
                        out_hbm.at[cid, sid])

    return agg(x, src3, dst3, w3, zeros)


def _tc_body(p_ref, w_ref, o_ref):
    s = p_ref[0] + p_ref[1]
    o_ref[...] = jnp.maximum(
        jnp.dot(s, w_ref[...], preferred_element_type=jnp.float32), 0.0)


_BM = 1000


def _tc_combine(partial, W):
    return pl.pallas_call(
        _tc_body,
        grid=(N // _BM,),
        in_specs=[
            pl.BlockSpec((NC, _BM, D), lambda i: (0, i, 0)),
            pl.BlockSpec((D, D), lambda i: (0, 0)),
        ],
        out_specs=pl.BlockSpec((_BM, D), lambda i: (i, 0)),
        out_shape=jax.ShapeDtypeStruct((N, D), jnp.float32),
    )(partial, W)


@jax.jit
def kernel(x, edge_index, edge_weight, W):
    pad = NW * EPT - E
    fill = jnp.arange(pad, dtype=jnp.int32) * 31 % N
    src = jnp.concatenate([edge_index[0], fill])
    dst = jnp.concatenate([edge_index[1], fill])
    w = jnp.pad(edge_weight, (0, pad))
    src3 = src.reshape(NW, NSB, SB, CH)
    dst3 = dst.reshape(NW, NSB, SB, CH)
    w3 = w.reshape(NW, NSB, SB, CH)
    zeros = jnp.zeros((RPT, D), jnp.float32)
    partial = _sc_aggregate(x, src3, dst3, w3, zeros).reshape(NC, N, D)
    return _tc_combine(partial, W)


# speedup vs baseline: 1.0839x; 1.0839x over previous
import functools

import jax
import jax.numpy as jnp
from jax import lax
from jax.experimental import pallas as pl
from jax.experimental.pallas import tpu as pltpu
from jax.experimental.pallas import tpu_sc as plsc

N = 10000
E = 320000
D = 128

_INFO = plsc.get_sparse_core_info()
NC = _INFO.num_cores
NS = _INFO.num_subcores
NW = NC * NS
CH = 100
SB = 20
NSB = 5
EPT = NSB * SB * CH
RPT = N // NS
NBUF = 3


def _scale_rows(buf, w_v, g):
    def row_body(r, c):
        q16 = jnp.minimum((r // 16) * 16, CH - 16)
        wrow = w_v[g, pl.ds(q16, 16)]
        idx = (jnp.zeros((16,), jnp.int32) + (r - q16))[:, None]
        wv = lax.gather(
            wrow, idx,
            lax.GatherDimensionNumbers(offset_dims=(),
                                       collapsed_slice_dims=(0,),
                                       start_index_map=(0,)),
            slice_sizes=(1,),
            mode=lax.GatherScatterMode.PROMISE_IN_BOUNDS)
        for j in range(D // 16):
            sl = (r, pl.ds(j * 16, 16))
            buf[sl] = buf[sl] * wv
        return c

    lax.fori_loop(0, CH, row_body, 0, unroll=False)


def _sc_aggregate(x, src3, dst3, w3, zeros):
    mesh = plsc.VectorSubcoreMesh(core_axis_name="c", subcore_axis_name="s")

    @functools.partial(
        pl.kernel,
        mesh=mesh,
        out_type=jax.ShapeDtypeStruct((NC, NS, RPT, D), jnp.float32),
        scratch_types=[
            pltpu.VMEM((SB, CH), jnp.int32),
            pltpu.VMEM((SB, CH), jnp.int32),
            pltpu.VMEM((SB, CH), jnp.float32),
            pltpu.VMEM((CH, D), jnp.float32),
            pltpu.VMEM((CH, D), jnp.float32),
            pltpu.VMEM((CH, D), jnp.float32),
            pltpu.VMEM_SHARED((N, D), jnp.float32),
            pltpu.SemaphoreType.DMA,
            pltpu.SemaphoreType.DMA,
            pltpu.SemaphoreType.DMA,
            pltpu.SemaphoreType.DMA,
            pltpu.SemaphoreType.DMA,
            pltpu.SemaphoreType.DMA,
        ],
    )
    def agg(x_hbm, src_hbm, dst_hbm, w_hbm, zeros_hbm, out_hbm,
            src_v, dst_v, w_v, buf0, buf1, buf2, acc,
            g0, g1, g2, s0, s1, s2):
        cid = lax.axis_index("c")
        sid = lax.axis_index("s")
        wid = sid * NC + cid
        bufs = (buf0, buf1, buf2)
        gsems = (g0, g1, g2)
        ssems = (s0, s1, s2)

        pltpu.sync_copy(zeros_hbm, acc.at[pl.ds(sid * RPT, RPT)])
        plsc.subcore_barrier()

        def super_body(sb, carry0):
            pltpu.sync_copy(src_hbm.at[wid, sb], src_v)
            pltpu.sync_copy(dst_hbm.at[wid, sb], dst_v)
            pltpu.sync_copy(w_hbm.at[wid, sb], w_v)

            gd = [None] * SB
            sd = [None] * SB
            for g in range(2):
                gd[g] = pltpu.async_copy(
                    x_hbm.at[src_v.at[g]], bufs[g % NBUF], gsems[g % NBUF])

            for g in range(SB):
                p = g % NBUF
                gd[g].wait()
                _scale_rows(bufs[p], w_v, g)
                if g + 2 < SB:
                    if g >= NBUF - 2:
                        sd[g - (NBUF - 2)].wait()
                    pn = (g + 2) % NBUF
                    gd[g + 2] = pltpu.async_copy(
                        x_hbm.at[src_v.at[g + 2]], bufs[pn], gsems[pn])
                sd[g] = pltpu.async_copy(
                    bufs[p], acc.at[dst_v.at[g]], ssems[p], add=True)
            for g in range(max(0, SB - NBUF), SB):
                sd[g].wait()
            return carry0

        lax.fori_loop(0, NSB, super_body, 0, unroll=False)

        plsc.subcore_barrier()
        pltpu.sync_copy(acc.at[pl.ds(sid * RPT, RPT)],
                        out_hbm.at[cid, sid])

    return agg(x, src3, dst3, w3, zeros)


def _tc_body(p_ref, w_ref, o_ref):
    s = p_ref[0] + p_ref[1]
    o_ref[...] = jnp.maximum(
        jnp.dot(s, w_ref[...], preferred_element_type=jnp.float32), 0.0)


_BM = 1000


def _tc_combine(partial, W):
    return pl.pallas_call(
        _tc_body,
        grid=(N // _BM,),
        in_specs=[
            pl.BlockSpec((NC, _BM, D), lambda i: (0, i, 0)),
            pl.BlockSpec((D, D), lambda i: (0, 0)),
        ],
        out_specs=pl.BlockSpec((_BM, D), lambda i: (i, 0)),
        out_shape=jax.ShapeDtypeStruct((N, D), jnp.float32),
    )(partial, W)


@jax.jit
def kernel(x, edge_index, edge_weight, W):
    pad = NW * EPT - E
    fill = jnp.arange(pad, dtype=jnp.int32) * 31 % N
    src = jnp.concatenate([edge_index[0], fill])
    dst = jnp.concatenate([edge_index[1], fill])
    w = jnp.pad(edge_weight, (0, pad))
    src3 = src.reshape(NW, NSB, SB, CH)
    dst3 = dst.reshape(NW, NSB, SB, CH)
    w3 = w.reshape(NW, NSB, SB, CH)
    zeros = jnp.zeros((RPT, D), jnp.float32)
    partial = _sc_aggregate(x, src3, dst3, w3, zeros).reshape(NC, N, D)
    return _tc_combine(partial, W)
